# Initial kernel scaffold; baseline (speedup 1.0000x reference)
import jax
import jax.numpy as jnp
from jax.experimental import pallas as pl


def kernel(real, imag, graph_sigs, W_conv, b_conv, w_pool, b_pool, W_head, b_head):
    raise NotImplementedError("write your pallas kernel here")


# speedup vs baseline: 3.7073x; 3.7073x over previous
import functools

import jax
import jax.numpy as jnp
from jax import lax
from jax.experimental import pallas as pl

B, T, N, F_IN = 256, 8, 30, 128
NF, C = 64, 10
G = 32


def _body(real_ref, sigs_ref, wconv_ref, bconv_ref, wpool_ref, bpool_ref,
          whead_ref, bhead_ref, out_ref):
    sigs = sigs_ref[...]
    adj = jnp.sum(sigs, axis=1) * (1.0 / T)
    x = real_ref[...].reshape(G * N, F_IN)
    xw = jnp.dot(x, wconv_ref[...], preferred_element_type=jnp.float32,
                 precision=lax.Precision.HIGHEST)
    xw = xw.reshape(G, N, NF)
    agg = lax.dot_general(adj, xw, (((1,), (1,)), ((0,), (0,))),
                          preferred_element_type=jnp.float32,
                          precision=lax.Precision.HIGHEST)
    h = jnp.maximum(agg + bconv_ref[...].reshape(1, 1, NF), 0.0)
    s = jnp.dot(h.reshape(G * N, NF), wpool_ref[...],
                preferred_element_type=jnp.float32,
                precision=lax.Precision.HIGHEST)
    s = s.reshape(G, N) + bpool_ref[0, 0]
    logits = lax.dot_general(s, whead_ref[...], (((1,), (1,)), ((), ())),
                             preferred_element_type=jnp.float32,
                             precision=lax.Precision.HIGHEST)
    logits = logits + bhead_ref[...]
    m = jnp.max(logits, axis=1, keepdims=True)
    e = jnp.exp(logits - m)
    out_ref[...] = e / jnp.sum(e, axis=1, keepdims=True)


def kernel(real, imag, graph_sigs, W_conv, b_conv, w_pool, b_pool, W_head, b_head):
    del imag
    whead2 = W_head.reshape(C, N)
    grid = (B // G,)
    return pl.pallas_call(
        _body,
        grid=grid,
        in_specs=[
            pl.BlockSpec((G, N, F_IN), lambda i: (i, 0, 0)),
            pl.BlockSpec((G, T, N, N), lambda i: (i, 0, 0, 0)),
            pl.BlockSpec((F_IN, NF), lambda i: (0, 0)),
            pl.BlockSpec((1, NF), lambda i: (0, 0)),
            pl.BlockSpec((NF, 1), lambda i: (0, 0)),
            pl.BlockSpec((1, 1), lambda i: (0, 0)),
            pl.BlockSpec((C, N), lambda i: (0, 0)),
            pl.BlockSpec((1, C), lambda i: (0, 0)),
        ],
        out_specs=pl.BlockSpec((G, C), lambda i: (i, 0)),
        out_shape=jax.ShapeDtypeStruct((B, C), jnp.float32),
    )(real, graph_sigs, W_conv, b_conv.reshape(1, NF), w_pool,
      b_pool.reshape(1, 1), whead2, b_head.reshape(1, C))
